# baseline (speedup 1.0000x reference)
import functools

import jax
import jax.numpy as jnp
from jax import lax
from jax.experimental import pallas as pl
from jax.experimental.pallas import tpu as pltpu
from jax.experimental.pallas import tpu_sc as plsc

_B = 4096
_L = 200
_N = _B * _L
_V = 100001
_T1 = 12
_T2 = 5
_D = 16
_LANES = 16

_NC = 2
_NS = 16

_W = 128
_VP = 100096
_NCHUNK = _VP // _W
_NSLOT = 2 * ((_NCHUNK + 2 * _NS - 1) // (2 * _NS))

_TPT = _N // _NS
_M = 1024
_GSTEPS = _TPT // _M


def _transpose_phase(tab_hbm, dst_hbm, in_bufs, out_bufs, sem_in, sem_out,
                     t_cnt, sid):
    nrows = t_cnt * _LANES
    iota = lax.broadcasted_iota(jnp.int32, (_LANES,), 0)

    nq = nrows // 4

    def start_in(k, b):
        for q in range(4):
            pltpu.async_copy(
                tab_hbm.at[pl.ds(q * nq, nq), pl.ds(k * _W, _W)],
                in_bufs[b].at[pl.ds(q * nq, nq), :], sem_in)

    def wait_in(k, b):
        for q in range(4):
            pltpu.make_async_copy(
                tab_hbm.at[pl.ds(q * nq, nq), pl.ds(k * _W, _W)],
                in_bufs[b].at[pl.ds(q * nq, nq), :], sem_in).wait()

    def out_slice(k):
        return dst_hbm.at[pl.ds(k * _W * t_cnt, _W * t_cnt)]

    def out_buf(b):
        return out_bufs[b].at[pl.ds(0, _W * t_cnt)]

    def transpose_chunk(b):
        in_stage, out_stage = in_bufs[b], out_bufs[b]

        def per_t(t, c1):
            def per_g(g, c2):
                colbase = g * _LANES
                rowvec = (colbase + iota) * t_cnt + t
                vals = [in_stage[t * _LANES + i, pl.ds(colbase, _LANES)]
                        for i in range(_LANES)]
                for i in range(_LANES):
                    plsc.store_scatter(
                        out_stage,
                        [rowvec, jnp.full((_LANES,), i, jnp.int32)], vals[i])
                return c2

            lax.fori_loop(0, _W // _LANES, per_g, 0)
            return c1

        lax.fori_loop(0, t_cnt, per_t, 0)

    start_in(sid, 0)

    def ring(kk2, c):
        for b in (0, 1):
            kk = kk2 * 2 + b
            k = sid + kk * _NS

            @pl.when(k < _NCHUNK)
            def _():
                wait_in(k, b)

                @pl.when(sid + (kk + 1) * _NS < _NCHUNK)
                def _():
                    start_in(sid + (kk + 1) * _NS, 1 - b)

                @pl.when(kk >= 2)
                def _():
                    pltpu.make_async_copy(
                        out_buf(b), out_slice(k), sem_out).wait()

                transpose_chunk(b)
                pltpu.async_copy(out_buf(b), out_slice(k), sem_out)

        return c

    lax.fori_loop(0, _NSLOT // 2, ring, 0)
    pltpu.make_async_copy(out_buf(0), out_slice(0), sem_out).wait()
    pltpu.make_async_copy(out_buf(1), out_slice(0), sem_out).wait()


def _gather_phase(src_hbm, ids_hbm, sel_hbm, out_hbm, ids_v, sel_v, idx_v,
                  rows_v, sem, t_cnt, col0, sid):
    def step(m, c):
        base = sid * _TPT + m * _M
        pltpu.sync_copy(ids_hbm.at[pl.ds(base, _M)], ids_v)
        pltpu.sync_copy(sel_hbm.at[pl.ds(base, _M)], sel_v)

        def compute(i, c2):
            s = pl.ds(i * _LANES, _LANES)
            idx_v[s] = ids_v[s] * t_cnt + sel_v[s]
            return c2

        lax.fori_loop(0, _M // _LANES, compute, 0)
        pltpu.async_copy(src_hbm.at[idx_v], rows_v, sem).wait()
        pltpu.sync_copy(rows_v, out_hbm.at[pl.ds(base, _M), pl.ds(col0, _D)])
        return c

    lax.fori_loop(0, _GSTEPS, step, 0)


@functools.partial(
    pl.kernel,
    out_type=jax.ShapeDtypeStruct((_N, 2 * _D), jnp.float32),
    mesh=plsc.VectorSubcoreMesh(
        core_axis_name="c", subcore_axis_name="s",
        num_cores=_NC, num_subcores=_NS),
    compiler_params=pltpu.CompilerParams(
        use_tc_tiling_on_sc=False, needs_layout_passes=False),
    scratch_types=[
        pltpu.HBM((_VP * _T1, _D), jnp.float32),
        pltpu.HBM((_VP * _T2, _D), jnp.float32),
        pltpu.VMEM((_T1 * _LANES, _W), jnp.float32),
        pltpu.VMEM((_T1 * _LANES, _W), jnp.float32),
        pltpu.VMEM((_W * _T1, _D), jnp.float32),
        pltpu.VMEM((_W * _T1, _D), jnp.float32),
        pltpu.VMEM((_M,), jnp.int32),
        pltpu.VMEM((_M,), jnp.int32),
        pltpu.VMEM((_M,), jnp.int32),
        pltpu.VMEM((_M, _D), jnp.float32),
        pltpu.SemaphoreType.DMA,
        pltpu.SemaphoreType.DMA,
        pltpu.SemaphoreType.DMA,
    ],
)
def _popularity_gather(log_hbm, t1_hbm, t2_hbm, mtab_hbm, wtab_hbm, out_hbm,
                       mt_hbm, wt_hbm, in0, in1, ost0, ost1,
                       ids_v, sel_v, idx_v, rows_v, sem, sem_in, sem_out):
    cid = lax.axis_index("c")
    sid = lax.axis_index("s")

    @pl.when(cid == 0)
    def _():
        _transpose_phase(mtab_hbm, mt_hbm, (in0, in1), (ost0, ost1),
                         sem_in, sem_out, _T1, sid)

    @pl.when(cid == 1)
    def _():
        _transpose_phase(wtab_hbm, wt_hbm, (in0, in1), (ost0, ost1),
                         sem_in, sem_out, _T2, sid)

    plsc.subcore_barrier()

    @pl.when(cid == 0)
    def _():
        _gather_phase(mt_hbm, log_hbm, t1_hbm, out_hbm, ids_v, sel_v, idx_v,
                      rows_v, sem, _T1, 0, sid)

    @pl.when(cid == 1)
    def _():
        _gather_phase(wt_hbm, log_hbm, t2_hbm, out_hbm, ids_v, sel_v, idx_v,
                      rows_v, sem, _T2, _D, sid)


def kernel(log_seqs, time1_seqs, time2_seqs, month_pop_table, week_pop_table):
    log = log_seqs.reshape(_N).astype(jnp.int32)
    t1 = time1_seqs.reshape(_N).astype(jnp.int32)
    t2 = time2_seqs.reshape(_N).astype(jnp.int32)
    mtab = jnp.pad(month_pop_table, ((0, 0), (0, _VP - _V)))
    wtab = jnp.pad(week_pop_table, ((0, 0), (0, _VP - _V)))
    out = _popularity_gather(log, t1, t2, mtab, wtab)
    return out.reshape(_B, _L, 2 * _D)


# speedup vs baseline: 1.0007x; 1.0007x over previous
import functools

import jax
import jax.numpy as jnp
from jax import lax
from jax.experimental import pallas as pl
from jax.experimental.pallas import tpu as pltpu
from jax.experimental.pallas import tpu_sc as plsc

_B = 4096
_L = 200
_N = _B * _L
_V = 100001
_T1 = 12
_T2 = 5
_D = 16
_LANES = 16

_NC = 2
_NS = 16

_W = 128
_VP = 100096
_NCHUNK = _VP // _W
_NSLOT = 2 * ((_NCHUNK + 2 * _NS - 1) // (2 * _NS))

_TPT = _N // _NS
_M = 1024
_GSTEPS = _TPT // _M


def _transpose_phase(tab_hbm, dst_hbm, in_bufs, out_bufs, sem_in, sem_out,
                     t_cnt, sid):
    nrows = t_cnt * _LANES
    iota = lax.broadcasted_iota(jnp.int32, (_LANES,), 0)

    nq = nrows // 4

    def start_in(k, b):
        for q in range(4):
            pltpu.async_copy(
                tab_hbm.at[pl.ds(q * nq, nq), pl.ds(k * _W, _W)],
                in_bufs[b].at[pl.ds(q * nq, nq), :], sem_in)

    def wait_in(k, b):
        for q in range(4):
            pltpu.make_async_copy(
                tab_hbm.at[pl.ds(q * nq, nq), pl.ds(k * _W, _W)],
                in_bufs[b].at[pl.ds(q * nq, nq), :], sem_in).wait()

    def out_slice(k):
        return dst_hbm.at[pl.ds(k * _W * t_cnt, _W * t_cnt)]

    def out_buf(b):
        return out_bufs[b].at[pl.ds(0, _W * t_cnt)]

    def transpose_chunk(b):
        in_stage, out_stage = in_bufs[b], out_bufs[b]
        rvs = [(g * _LANES + iota) * t_cnt for g in range(_W // _LANES)]
        cols = [jnp.full((_LANES,), i, jnp.int32) for i in range(_LANES)]

        def per_t(t, c1):
            for g in range(_W // _LANES):
                rowvec = rvs[g] + t
                vals = [in_stage[t * _LANES + i, pl.ds(g * _LANES, _LANES)]
                        for i in range(_LANES)]
                for i in range(_LANES):
                    plsc.store_scatter(out_stage, [rowvec, cols[i]], vals[i])
            return c1

        lax.fori_loop(0, t_cnt, per_t, 0)

    start_in(sid, 0)

    def ring(kk2, c):
        for b in (0, 1):
            kk = kk2 * 2 + b
            k = sid + kk * _NS

            @pl.when(k < _NCHUNK)
            def _():
                wait_in(k, b)

                @pl.when(sid + (kk + 1) * _NS < _NCHUNK)
                def _():
                    start_in(sid + (kk + 1) * _NS, 1 - b)

                @pl.when(kk >= 2)
                def _():
                    pltpu.make_async_copy(
                        out_buf(b), out_slice(k), sem_out).wait()

                transpose_chunk(b)
                pltpu.async_copy(out_buf(b), out_slice(k), sem_out)

        return c

    lax.fori_loop(0, _NSLOT // 2, ring, 0)
    pltpu.make_async_copy(out_buf(0), out_slice(0), sem_out).wait()
    pltpu.make_async_copy(out_buf(1), out_slice(0), sem_out).wait()


def _gather_phase(src_hbm, ids_hbm, sel_hbm, out_hbm, ids_v, sel_v, idx_v,
                  rows_v, sem, t_cnt, col0, sid):
    def step(m, c):
        base = sid * _TPT + m * _M
        pltpu.sync_copy(ids_hbm.at[pl.ds(base, _M)], ids_v)
        pltpu.sync_copy(sel_hbm.at[pl.ds(base, _M)], sel_v)

        def compute(i, c2):
            s = pl.ds(i * _LANES, _LANES)
            idx_v[s] = ids_v[s] * t_cnt + sel_v[s]
            return c2

        lax.fori_loop(0, _M // _LANES, compute, 0)
        pltpu.async_copy(src_hbm.at[idx_v], rows_v, sem).wait()
        pltpu.sync_copy(rows_v, out_hbm.at[pl.ds(base, _M), pl.ds(col0, _D)])
        return c

    lax.fori_loop(0, _GSTEPS, step, 0)


@functools.partial(
    pl.kernel,
    out_type=jax.ShapeDtypeStruct((_N, 2 * _D), jnp.float32),
    mesh=plsc.VectorSubcoreMesh(
        core_axis_name="c", subcore_axis_name="s",
        num_cores=_NC, num_subcores=_NS),
    compiler_params=pltpu.CompilerParams(
        use_tc_tiling_on_sc=False, needs_layout_passes=False),
    scratch_types=[
        pltpu.HBM((_VP * _T1, _D), jnp.float32),
        pltpu.HBM((_VP * _T2, _D), jnp.float32),
        pltpu.VMEM((_T1 * _LANES, _W), jnp.float32),
        pltpu.VMEM((_T1 * _LANES, _W), jnp.float32),
        pltpu.VMEM((_W * _T1, _D), jnp.float32),
        pltpu.VMEM((_W * _T1, _D), jnp.float32),
        pltpu.VMEM((_M,), jnp.int32),
        pltpu.VMEM((_M,), jnp.int32),
        pltpu.VMEM((_M,), jnp.int32),
        pltpu.VMEM((_M, _D), jnp.float32),
        pltpu.SemaphoreType.DMA,
        pltpu.SemaphoreType.DMA,
        pltpu.SemaphoreType.DMA,
    ],
)
def _popularity_gather(log_hbm, t1_hbm, t2_hbm, mtab_hbm, wtab_hbm, out_hbm,
                       mt_hbm, wt_hbm, in0, in1, ost0, ost1,
                       ids_v, sel_v, idx_v, rows_v, sem, sem_in, sem_out):
    cid = lax.axis_index("c")
    sid = lax.axis_index("s")

    @pl.when(cid == 0)
    def _():
        _transpose_phase(mtab_hbm, mt_hbm, (in0, in1), (ost0, ost1),
                         sem_in, sem_out, _T1, sid)

    @pl.when(cid == 1)
    def _():
        _transpose_phase(wtab_hbm, wt_hbm, (in0, in1), (ost0, ost1),
                         sem_in, sem_out, _T2, sid)

    plsc.subcore_barrier()

    @pl.when(cid == 0)
    def _():
        _gather_phase(mt_hbm, log_hbm, t1_hbm, out_hbm, ids_v, sel_v, idx_v,
                      rows_v, sem, _T1, 0, sid)

    @pl.when(cid == 1)
    def _():
        _gather_phase(wt_hbm, log_hbm, t2_hbm, out_hbm, ids_v, sel_v, idx_v,
                      rows_v, sem, _T2, _D, sid)


def kernel(log_seqs, time1_seqs, time2_seqs, month_pop_table, week_pop_table):
    log = log_seqs.reshape(_N).astype(jnp.int32)
    t1 = time1_seqs.reshape(_N).astype(jnp.int32)
    t2 = time2_seqs.reshape(_N).astype(jnp.int32)
    mtab = jnp.pad(month_pop_table, ((0, 0), (0, _VP - _V)))
    wtab = jnp.pad(week_pop_table, ((0, 0), (0, _VP - _V)))
    out = _popularity_gather(log, t1, t2, mtab, wtab)
    return out.reshape(_B, _L, 2 * _D)


# speedup vs baseline: 1.0137x; 1.0130x over previous
import functools

import jax
import jax.numpy as jnp
from jax import lax
from jax.experimental import pallas as pl
from jax.experimental.pallas import tpu as pltpu
from jax.experimental.pallas import tpu_sc as plsc

_B = 4096
_L = 200
_N = _B * _L
_V = 100001
_T1 = 12
_T2 = 5
_D = 16
_LANES = 16

_NC = 2
_NS = 16

_W = 128
_VP = 100096
_NCHUNK = _VP // _W
_NSLOT = 2 * ((_NCHUNK + 2 * _NS - 1) // (2 * _NS))

_TPT = _N // _NS
_M = 1024
_GSTEPS = _TPT // _M


def _transpose_phase(tab_hbm, dst_hbm, in_bufs, out_bufs, sem_in, sem_out,
                     t_cnt, sid):
    nrows = t_cnt * _LANES
    iota = lax.broadcasted_iota(jnp.int32, (_LANES,), 0)

    nq = nrows // 4

    def start_in(k, b):
        for q in range(4):
            pltpu.async_copy(
                tab_hbm.at[pl.ds(q * nq, nq), pl.ds(k * _W, _W)],
                in_bufs[b].at[pl.ds(q * nq, nq), pl.ds(0, _W)], sem_in)

    def wait_in(k, b):
        for q in range(4):
            pltpu.make_async_copy(
                tab_hbm.at[pl.ds(q * nq, nq), pl.ds(k * _W, _W)],
                in_bufs[b].at[pl.ds(q * nq, nq), pl.ds(0, _W)], sem_in).wait()

    def out_slice(k):
        return dst_hbm.at[pl.ds(k * _W * t_cnt, _W * t_cnt)]

    def out_buf(b):
        return out_bufs[b].at[pl.ds(0, _W * t_cnt)]

    def transpose_chunk(b):
        in_stage, out_stage = in_bufs[b], out_bufs[b]

        def per_t(t, c1):
            rowidx = t * _LANES + iota
            for c in range(_W):
                v = plsc.load_gather(
                    in_stage, [rowidx, jnp.full((_LANES,), c, jnp.int32)])
                out_stage[c * t_cnt + t, :] = v
            return c1

        lax.fori_loop(0, t_cnt, per_t, 0)

    start_in(sid, 0)

    def ring(kk2, c):
        for b in (0, 1):
            kk = kk2 * 2 + b
            k = sid + kk * _NS

            @pl.when(k < _NCHUNK)
            def _():
                wait_in(k, b)

                @pl.when(sid + (kk + 1) * _NS < _NCHUNK)
                def _():
                    start_in(sid + (kk + 1) * _NS, 1 - b)

                @pl.when(kk >= 2)
                def _():
                    pltpu.make_async_copy(
                        out_buf(b), out_slice(k), sem_out).wait()

                transpose_chunk(b)
                pltpu.async_copy(out_buf(b), out_slice(k), sem_out)

        return c

    lax.fori_loop(0, _NSLOT // 2, ring, 0)
    pltpu.make_async_copy(out_buf(0), out_slice(0), sem_out).wait()
    pltpu.make_async_copy(out_buf(1), out_slice(0), sem_out).wait()


def _gather_phase(src_hbm, ids_hbm, sel_hbm, out_hbm, ids_v, sel_v, idx_v,
                  rows_v, sem, t_cnt, col0, sid):
    def step(m, c):
        base = sid * _TPT + m * _M
        pltpu.sync_copy(ids_hbm.at[pl.ds(base, _M)], ids_v)
        pltpu.sync_copy(sel_hbm.at[pl.ds(base, _M)], sel_v)

        def compute(i, c2):
            s = pl.ds(i * _LANES, _LANES)
            idx_v[s] = ids_v[s] * t_cnt + sel_v[s]
            return c2

        lax.fori_loop(0, _M // _LANES, compute, 0)
        pltpu.async_copy(src_hbm.at[idx_v], rows_v, sem).wait()
        pltpu.sync_copy(rows_v, out_hbm.at[pl.ds(base, _M), pl.ds(col0, _D)])
        return c

    lax.fori_loop(0, _GSTEPS, step, 0)


@functools.partial(
    pl.kernel,
    out_type=jax.ShapeDtypeStruct((_N, 2 * _D), jnp.float32),
    mesh=plsc.VectorSubcoreMesh(
        core_axis_name="c", subcore_axis_name="s",
        num_cores=_NC, num_subcores=_NS),
    compiler_params=pltpu.CompilerParams(
        use_tc_tiling_on_sc=False, needs_layout_passes=False),
    scratch_types=[
        pltpu.HBM((_VP * _T1, _D), jnp.float32),
        pltpu.HBM((_VP * _T2, _D), jnp.float32),
        pltpu.VMEM((_T1 * _LANES, _W + 1), jnp.float32),
        pltpu.VMEM((_T1 * _LANES, _W + 1), jnp.float32),
        pltpu.VMEM((_W * _T1, _D), jnp.float32),
        pltpu.VMEM((_W * _T1, _D), jnp.float32),
        pltpu.VMEM((_M,), jnp.int32),
        pltpu.VMEM((_M,), jnp.int32),
        pltpu.VMEM((_M,), jnp.int32),
        pltpu.VMEM((_M, _D), jnp.float32),
        pltpu.SemaphoreType.DMA,
        pltpu.SemaphoreType.DMA,
        pltpu.SemaphoreType.DMA,
    ],
)
def _popularity_gather(log_hbm, t1_hbm, t2_hbm, mtab_hbm, wtab_hbm, out_hbm,
                       mt_hbm, wt_hbm, in0, in1, ost0, ost1,
                       ids_v, sel_v, idx_v, rows_v, sem, sem_in, sem_out):
    cid = lax.axis_index("c")
    sid = lax.axis_index("s")

    @pl.when(cid == 0)
    def _():
        _transpose_phase(mtab_hbm, mt_hbm, (in0, in1), (ost0, ost1),
                         sem_in, sem_out, _T1, sid)

    @pl.when(cid == 1)
    def _():
        _transpose_phase(wtab_hbm, wt_hbm, (in0, in1), (ost0, ost1),
                         sem_in, sem_out, _T2, sid)

    plsc.subcore_barrier()

    @pl.when(cid == 0)
    def _():
        _gather_phase(mt_hbm, log_hbm, t1_hbm, out_hbm, ids_v, sel_v, idx_v,
                      rows_v, sem, _T1, 0, sid)

    @pl.when(cid == 1)
    def _():
        _gather_phase(wt_hbm, log_hbm, t2_hbm, out_hbm, ids_v, sel_v, idx_v,
                      rows_v, sem, _T2, _D, sid)


def kernel(log_seqs, time1_seqs, time2_seqs, month_pop_table, week_pop_table):
    log = log_seqs.reshape(_N).astype(jnp.int32)
    t1 = time1_seqs.reshape(_N).astype(jnp.int32)
    t2 = time2_seqs.reshape(_N).astype(jnp.int32)
    mtab = jnp.pad(month_pop_table, ((0, 0), (0, _VP - _V)))
    wtab = jnp.pad(week_pop_table, ((0, 0), (0, _VP - _V)))
    out = _popularity_gather(log, t1, t2, mtab, wtab)
    return out.reshape(_B, _L, 2 * _D)


# speedup vs baseline: 1.4444x; 1.4249x over previous
import functools

import jax
import jax.numpy as jnp
from jax import lax
from jax.experimental import pallas as pl
from jax.experimental.pallas import tpu as pltpu
from jax.experimental.pallas import tpu_sc as plsc

_B = 4096
_L = 200
_N = _B * _L
_V = 100001
_T1 = 12
_T2 = 5
_D = 16
_LANES = 16

_NC = 2
_NS = 16

_W = 128
_VP = 100096
_NCHUNK = _VP // _W
_NSLOT = 2 * ((_NCHUNK + 2 * _NS - 1) // (2 * _NS))

_TPT = _N // _NS
_M = 1280
_GSTEPS = _TPT // _M


def _transpose_phase(tab_hbm, dst_hbm, in_bufs, out_bufs, sem_in, sem_out,
                     t_cnt, sid):
    nrows = t_cnt * _LANES
    iota = lax.broadcasted_iota(jnp.int32, (_LANES,), 0)

    nq = nrows // 4

    def start_in(k, b):
        for q in range(4):
            pltpu.async_copy(
                tab_hbm.at[pl.ds(q * nq, nq), pl.ds(k * _W, _W)],
                in_bufs[b].at[pl.ds(q * nq, nq), pl.ds(0, _W)], sem_in)

    def wait_in(k, b):
        for q in range(4):
            pltpu.make_async_copy(
                tab_hbm.at[pl.ds(q * nq, nq), pl.ds(k * _W, _W)],
                in_bufs[b].at[pl.ds(q * nq, nq), pl.ds(0, _W)], sem_in).wait()

    def out_slice(k):
        return dst_hbm.at[pl.ds(k * _W * t_cnt, _W * t_cnt)]

    def out_buf(b):
        return out_bufs[b].at[pl.ds(0, _W * t_cnt)]

    def transpose_chunk(b):
        in_stage, out_stage = in_bufs[b], out_bufs[b]

        def per_t(t, c1):
            rowidx = t * _LANES + iota
            for c0 in range(0, _W, 8):
                vs = [plsc.load_gather(
                    in_stage,
                    [rowidx, jnp.full((_LANES,), c0 + j, jnp.int32)])
                    for j in range(8)]
                for j in range(8):
                    out_stage[(c0 + j) * t_cnt + t, :] = vs[j]
            return c1

        lax.fori_loop(0, t_cnt, per_t, 0)

    start_in(sid, 0)

    def ring(kk2, c):
        for b in (0, 1):
            kk = kk2 * 2 + b
            k = sid + kk * _NS

            @pl.when(k < _NCHUNK)
            def _():
                wait_in(k, b)

                @pl.when(sid + (kk + 1) * _NS < _NCHUNK)
                def _():
                    start_in(sid + (kk + 1) * _NS, 1 - b)

                @pl.when(kk >= 2)
                def _():
                    pltpu.make_async_copy(
                        out_buf(b), out_slice(k), sem_out).wait()

                transpose_chunk(b)
                pltpu.async_copy(out_buf(b), out_slice(k), sem_out)

        return c

    lax.fori_loop(0, _NSLOT // 2, ring, 0)
    pltpu.make_async_copy(out_buf(0), out_slice(0), sem_out).wait()
    pltpu.make_async_copy(out_buf(1), out_slice(0), sem_out).wait()


def _gather_phase(src_hbm, ids_hbm, sel_hbm, out_hbm, ids_v, sel_v, idx_v,
                  rows_v, sem, t_cnt, col0, sid):
    def step(m, c):
        base = sid * _TPT + m * _M
        pltpu.sync_copy(ids_hbm.at[pl.ds(base, _M)], ids_v)
        pltpu.sync_copy(sel_hbm.at[pl.ds(base, _M)], sel_v)

        def compute(i, c2):
            s = pl.ds(i * _LANES, _LANES)
            idx_v[s] = ids_v[s] * t_cnt + sel_v[s]
            return c2

        lax.fori_loop(0, _M // _LANES, compute, 0)
        pltpu.async_copy(src_hbm.at[idx_v], rows_v, sem).wait()
        pltpu.sync_copy(rows_v, out_hbm.at[pl.ds(base, _M), pl.ds(col0, _D)])
        return c

    lax.fori_loop(0, _GSTEPS, step, 0)


@functools.partial(
    pl.kernel,
    out_type=jax.ShapeDtypeStruct((_N, 2 * _D), jnp.float32),
    mesh=plsc.VectorSubcoreMesh(
        core_axis_name="c", subcore_axis_name="s",
        num_cores=_NC, num_subcores=_NS),
    compiler_params=pltpu.CompilerParams(
        use_tc_tiling_on_sc=False, needs_layout_passes=False),
    scratch_types=[
        pltpu.HBM((_VP * _T1, _D), jnp.float32),
        pltpu.HBM((_VP * _T2, _D), jnp.float32),
        pltpu.VMEM((_T1 * _LANES, _W + 1), jnp.float32),
        pltpu.VMEM((_T1 * _LANES, _W + 1), jnp.float32),
        pltpu.VMEM((_W * _T1, _D), jnp.float32),
        pltpu.VMEM((_W * _T1, _D), jnp.float32),
        pltpu.VMEM((_M,), jnp.int32),
        pltpu.VMEM((_M,), jnp.int32),
        pltpu.VMEM((_M,), jnp.int32),
        pltpu.VMEM((_M, _D), jnp.float32),
        pltpu.SemaphoreType.DMA,
        pltpu.SemaphoreType.DMA,
        pltpu.SemaphoreType.DMA,
    ],
)
def _popularity_gather(log_hbm, t1_hbm, t2_hbm, mtab_hbm, wtab_hbm, out_hbm,
                       mt_hbm, wt_hbm, in0, in1, ost0, ost1,
                       ids_v, sel_v, idx_v, rows_v, sem, sem_in, sem_out):
    cid = lax.axis_index("c")
    sid = lax.axis_index("s")

    @pl.when(cid == 0)
    def _():
        _transpose_phase(mtab_hbm, mt_hbm, (in0, in1), (ost0, ost1),
                         sem_in, sem_out, _T1, sid)

    @pl.when(cid == 1)
    def _():
        _transpose_phase(wtab_hbm, wt_hbm, (in0, in1), (ost0, ost1),
                         sem_in, sem_out, _T2, sid)

    plsc.subcore_barrier()

    @pl.when(cid == 0)
    def _():
        _gather_phase(mt_hbm, log_hbm, t1_hbm, out_hbm, ids_v, sel_v, idx_v,
                      rows_v, sem, _T1, 0, sid)

    @pl.when(cid == 1)
    def _():
        _gather_phase(wt_hbm, log_hbm, t2_hbm, out_hbm, ids_v, sel_v, idx_v,
                      rows_v, sem, _T2, _D, sid)


def kernel(log_seqs, time1_seqs, time2_seqs, month_pop_table, week_pop_table):
    log = log_seqs.reshape(_N).astype(jnp.int32)
    t1 = time1_seqs.reshape(_N).astype(jnp.int32)
    t2 = time2_seqs.reshape(_N).astype(jnp.int32)
    mtab = jnp.pad(month_pop_table, ((0, 0), (0, _VP - _V)))
    wtab = jnp.pad(week_pop_table, ((0, 0), (0, _VP - _V)))
    out = _popularity_gather(log, t1, t2, mtab, wtab)
    return out.reshape(_B, _L, 2 * _D)


# speedup vs baseline: 1.4636x; 1.0133x over previous
import functools

import jax
import jax.numpy as jnp
from jax import lax
from jax.experimental import pallas as pl
from jax.experimental.pallas import tpu as pltpu
from jax.experimental.pallas import tpu_sc as plsc

_B = 4096
_L = 200
_N = _B * _L
_V = 100001
_T1 = 12
_T2 = 5
_D = 16
_LANES = 16

_NC = 2
_NS = 16

_W = 128
_VP = 100096
_NCHUNK = _VP // _W
_NSLOT = 2 * ((_NCHUNK + 2 * _NS - 1) // (2 * _NS))

_TPT = _N // _NS
_M = 1280
_GSTEPS = _TPT // _M


def _transpose_phase(tab_hbm, dst_hbm, in_bufs, out_bufs, sem_in, sem_out,
                     t_cnt, sid):
    nrows = t_cnt * _LANES
    iota = lax.broadcasted_iota(jnp.int32, (_LANES,), 0)

    nq = nrows // 4

    def start_in(k, b):
        for q in range(4):
            pltpu.async_copy(
                tab_hbm.at[pl.ds(q * nq, nq), pl.ds(k * _W, _W)],
                in_bufs[b].at[pl.ds(q * nq, nq), pl.ds(0, _W)], sem_in)

    def wait_in(k, b):
        for q in range(4):
            pltpu.make_async_copy(
                tab_hbm.at[pl.ds(q * nq, nq), pl.ds(k * _W, _W)],
                in_bufs[b].at[pl.ds(q * nq, nq), pl.ds(0, _W)], sem_in).wait()

    def out_slice(k):
        return dst_hbm.at[pl.ds(k * _W * t_cnt, _W * t_cnt)]

    def out_buf(b):
        return out_bufs[b].at[pl.ds(0, _W * t_cnt)]

    def transpose_chunk(b):
        in_stage, out_stage = in_bufs[b], out_bufs[b]

        def per_t(t, c1):
            rowidx = t * _LANES + iota
            for c0 in range(0, _W, 16):
                vs = [plsc.load_gather(
                    in_stage,
                    [rowidx, jnp.full((_LANES,), c0 + j, jnp.int32)])
                    for j in range(16)]
                for j in range(16):
                    out_stage[(c0 + j) * t_cnt + t, :] = vs[j]
            return c1

        lax.fori_loop(0, t_cnt, per_t, 0)

    start_in(sid, 0)

    def ring(kk2, c):
        for b in (0, 1):
            kk = kk2 * 2 + b
            k = sid + kk * _NS

            @pl.when(k < _NCHUNK)
            def _():
                wait_in(k, b)

                @pl.when(sid + (kk + 1) * _NS < _NCHUNK)
                def _():
                    start_in(sid + (kk + 1) * _NS, 1 - b)

                @pl.when(kk >= 2)
                def _():
                    pltpu.make_async_copy(
                        out_buf(b), out_slice(k), sem_out).wait()

                transpose_chunk(b)
                pltpu.async_copy(out_buf(b), out_slice(k), sem_out)

        return c

    lax.fori_loop(0, _NSLOT // 2, ring, 0)
    pltpu.make_async_copy(out_buf(0), out_slice(0), sem_out).wait()
    pltpu.make_async_copy(out_buf(1), out_slice(0), sem_out).wait()


def _gather_phase(src_hbm, ids_hbm, sel_hbm, out_hbm, ids_v, sel_v, idx_v,
                  rows_v, sem, t_cnt, col0, sid):
    def step(m, c):
        base = sid * _TPT + m * _M
        pltpu.sync_copy(ids_hbm.at[pl.ds(base, _M)], ids_v)
        pltpu.sync_copy(sel_hbm.at[pl.ds(base, _M)], sel_v)

        def compute(i, c2):
            s = pl.ds(i * _LANES, _LANES)
            idx_v[s] = ids_v[s] * t_cnt + sel_v[s]
            return c2

        lax.fori_loop(0, _M // _LANES, compute, 0)
        pltpu.async_copy(src_hbm.at[idx_v], rows_v, sem).wait()
        pltpu.sync_copy(rows_v, out_hbm.at[pl.ds(base, _M), pl.ds(col0, _D)])
        return c

    lax.fori_loop(0, _GSTEPS, step, 0)


@functools.partial(
    pl.kernel,
    out_type=jax.ShapeDtypeStruct((_N, 2 * _D), jnp.float32),
    mesh=plsc.VectorSubcoreMesh(
        core_axis_name="c", subcore_axis_name="s",
        num_cores=_NC, num_subcores=_NS),
    compiler_params=pltpu.CompilerParams(
        use_tc_tiling_on_sc=False, needs_layout_passes=False),
    scratch_types=[
        pltpu.HBM((_VP * _T1, _D), jnp.float32),
        pltpu.HBM((_VP * _T2, _D), jnp.float32),
        pltpu.VMEM((_T1 * _LANES, _W + 1), jnp.float32),
        pltpu.VMEM((_T1 * _LANES, _W + 1), jnp.float32),
        pltpu.VMEM((_W * _T1, _D), jnp.float32),
        pltpu.VMEM((_W * _T1, _D), jnp.float32),
        pltpu.VMEM((_M,), jnp.int32),
        pltpu.VMEM((_M,), jnp.int32),
        pltpu.VMEM((_M,), jnp.int32),
        pltpu.VMEM((_M, _D), jnp.float32),
        pltpu.SemaphoreType.DMA,
        pltpu.SemaphoreType.DMA,
        pltpu.SemaphoreType.DMA,
    ],
)
def _popularity_gather(log_hbm, t1_hbm, t2_hbm, mtab_hbm, wtab_hbm, out_hbm,
                       mt_hbm, wt_hbm, in0, in1, ost0, ost1,
                       ids_v, sel_v, idx_v, rows_v, sem, sem_in, sem_out):
    cid = lax.axis_index("c")
    sid = lax.axis_index("s")

    @pl.when(cid == 0)
    def _():
        _transpose_phase(mtab_hbm, mt_hbm, (in0, in1), (ost0, ost1),
                         sem_in, sem_out, _T1, sid)

    @pl.when(cid == 1)
    def _():
        _transpose_phase(wtab_hbm, wt_hbm, (in0, in1), (ost0, ost1),
                         sem_in, sem_out, _T2, sid)

    plsc.subcore_barrier()

    @pl.when(cid == 0)
    def _():
        _gather_phase(mt_hbm, log_hbm, t1_hbm, out_hbm, ids_v, sel_v, idx_v,
                      rows_v, sem, _T1, 0, sid)

    @pl.when(cid == 1)
    def _():
        _gather_phase(wt_hbm, log_hbm, t2_hbm, out_hbm, ids_v, sel_v, idx_v,
                      rows_v, sem, _T2, _D, sid)


def kernel(log_seqs, time1_seqs, time2_seqs, month_pop_table, week_pop_table):
    log = log_seqs.reshape(_N).astype(jnp.int32)
    t1 = time1_seqs.reshape(_N).astype(jnp.int32)
    t2 = time2_seqs.reshape(_N).astype(jnp.int32)
    mtab = jnp.pad(month_pop_table, ((0, 0), (0, _VP - _V)))
    wtab = jnp.pad(week_pop_table, ((0, 0), (0, _VP - _V)))
    out = _popularity_gather(log, t1, t2, mtab, wtab)
    return out.reshape(_B, _L, 2 * _D)


# speedup vs baseline: 1.5637x; 1.0684x over previous
import functools

import jax
import jax.numpy as jnp
from jax import lax
from jax.experimental import pallas as pl
from jax.experimental.pallas import tpu as pltpu
from jax.experimental.pallas import tpu_sc as plsc

_B = 4096
_L = 200
_N = _B * _L
_V = 100001
_T1 = 12
_T2 = 5
_D = 16
_LANES = 16

_NC = 2
_NS = 16

_W = 128
_VP = 100096
_NCHUNK = _VP // _W
_NSLOT = 2 * ((_NCHUNK + 2 * _NS - 1) // (2 * _NS))

_TPT = _N // _NS
_M = 640
_GSTEPS = _TPT // _M


def _transpose_phase(tab_hbm, dst_hbm, in_bufs, out_bufs, sem_in, sem_out,
                     t_cnt, sid):
    nrows = t_cnt * _LANES
    iota = lax.broadcasted_iota(jnp.int32, (_LANES,), 0)

    nq = nrows // 4

    def start_in(k, b):
        for q in range(4):
            pltpu.async_copy(
                tab_hbm.at[pl.ds(q * nq, nq), pl.ds(k * _W, _W)],
                in_bufs[b].at[pl.ds(q * nq, nq), pl.ds(0, _W)], sem_in)

    def wait_in(k, b):
        for q in range(4):
            pltpu.make_async_copy(
                tab_hbm.at[pl.ds(q * nq, nq), pl.ds(k * _W, _W)],
                in_bufs[b].at[pl.ds(q * nq, nq), pl.ds(0, _W)], sem_in).wait()

    def out_slice(k):
        return dst_hbm.at[pl.ds(k * _W * t_cnt, _W * t_cnt)]

    def out_buf(b):
        return out_bufs[b].at[pl.ds(0, _W * t_cnt)]

    def transpose_chunk(b):
        in_stage, out_stage = in_bufs[b], out_bufs[b]

        def per_t(t, c1):
            rowidx = t * _LANES + iota
            for c0 in range(0, _W, 16):
                vs = [plsc.load_gather(
                    in_stage,
                    [rowidx, jnp.full((_LANES,), c0 + j, jnp.int32)])
                    for j in range(16)]
                for j in range(16):
                    out_stage[(c0 + j) * t_cnt + t, :] = vs[j]
            return c1

        lax.fori_loop(0, t_cnt, per_t, 0)

    start_in(sid, 0)

    def ring(kk2, c):
        for b in (0, 1):
            kk = kk2 * 2 + b
            k = sid + kk * _NS

            @pl.when(k < _NCHUNK)
            def _():
                wait_in(k, b)

                @pl.when(sid + (kk + 1) * _NS < _NCHUNK)
                def _():
                    start_in(sid + (kk + 1) * _NS, 1 - b)

                @pl.when(kk >= 2)
                def _():
                    pltpu.make_async_copy(
                        out_buf(b), out_slice(k), sem_out).wait()

                transpose_chunk(b)
                pltpu.async_copy(out_buf(b), out_slice(k), sem_out)

        return c

    lax.fori_loop(0, _NSLOT // 2, ring, 0)
    pltpu.make_async_copy(out_buf(0), out_slice(0), sem_out).wait()
    pltpu.make_async_copy(out_buf(1), out_slice(0), sem_out).wait()


def _gather_phase(src_hbm, ids_hbm, sel_hbm, out_hbm, ids_bufs, sel_bufs,
                  idx_bufs, row_bufs, sem_g, sem_in, sem_out, t_cnt, col0,
                  sid):
    def start_in(m, b):
        base = sid * _TPT + m * _M
        pltpu.async_copy(ids_hbm.at[pl.ds(base, _M)], ids_bufs[b], sem_in)
        pltpu.async_copy(sel_hbm.at[pl.ds(base, _M)], sel_bufs[b], sem_in)

    def wait_in(m, b):
        base = sid * _TPT + m * _M
        pltpu.make_async_copy(
            ids_hbm.at[pl.ds(base, _M)], ids_bufs[b], sem_in).wait()
        pltpu.make_async_copy(
            sel_hbm.at[pl.ds(base, _M)], sel_bufs[b], sem_in).wait()

    def out_slice(m):
        base = sid * _TPT + m * _M
        return out_hbm.at[pl.ds(base, _M), pl.ds(col0, _D)]

    start_in(0, 0)

    def ring(m2, c):
        for b in (0, 1):
            m = m2 * 2 + b
            wait_in(m, b)

            @pl.when(m + 1 < _GSTEPS)
            def _():
                start_in(m + 1, 1 - b)

            def compute(i, c2):
                s = pl.ds(i * _LANES, _LANES)
                idx_bufs[b][s] = ids_bufs[b][s] * t_cnt + sel_bufs[b][s]
                return c2

            lax.fori_loop(0, _M // _LANES, compute, 0)

            @pl.when(m >= 2)
            def _():
                pltpu.make_async_copy(
                    row_bufs[b], out_slice(m), sem_out).wait()

            pltpu.async_copy(src_hbm.at[idx_bufs[b]], row_bufs[b], sem_g).wait()
            pltpu.async_copy(row_bufs[b], out_slice(m), sem_out)
        return c

    lax.fori_loop(0, _GSTEPS // 2, ring, 0)
    pltpu.make_async_copy(row_bufs[0], out_slice(0), sem_out).wait()
    pltpu.make_async_copy(row_bufs[1], out_slice(0), sem_out).wait()


@functools.partial(
    pl.kernel,
    out_type=jax.ShapeDtypeStruct((_N, 2 * _D), jnp.float32),
    mesh=plsc.VectorSubcoreMesh(
        core_axis_name="c", subcore_axis_name="s",
        num_cores=_NC, num_subcores=_NS),
    compiler_params=pltpu.CompilerParams(
        use_tc_tiling_on_sc=False, needs_layout_passes=False),
    scratch_types=[
        pltpu.HBM((_VP * _T1, _D), jnp.float32),
        pltpu.HBM((_VP * _T2, _D), jnp.float32),
        pltpu.VMEM((_T1 * _LANES, _W + 1), jnp.float32),
        pltpu.VMEM((_T1 * _LANES, _W + 1), jnp.float32),
        pltpu.VMEM((_W * _T1, _D), jnp.float32),
        pltpu.VMEM((_W * _T1, _D), jnp.float32),
        pltpu.VMEM((_M,), jnp.int32),
        pltpu.VMEM((_M,), jnp.int32),
        pltpu.VMEM((_M,), jnp.int32),
        pltpu.VMEM((_M,), jnp.int32),
        pltpu.VMEM((_M,), jnp.int32),
        pltpu.VMEM((_M,), jnp.int32),
        pltpu.VMEM((_M, _D), jnp.float32),
        pltpu.VMEM((_M, _D), jnp.float32),
        pltpu.SemaphoreType.DMA,
        pltpu.SemaphoreType.DMA,
        pltpu.SemaphoreType.DMA,
    ],
)
def _popularity_gather(log_hbm, t1_hbm, t2_hbm, mtab_hbm, wtab_hbm, out_hbm,
                       mt_hbm, wt_hbm, in0, in1, ost0, ost1,
                       ids0, ids1, sel0, sel1, idx0, idx1, rows0, rows1,
                       sem, sem_in, sem_out):
    cid = lax.axis_index("c")
    sid = lax.axis_index("s")

    @pl.when(cid == 0)
    def _():
        _transpose_phase(mtab_hbm, mt_hbm, (in0, in1), (ost0, ost1),
                         sem_in, sem_out, _T1, sid)

    @pl.when(cid == 1)
    def _():
        _transpose_phase(wtab_hbm, wt_hbm, (in0, in1), (ost0, ost1),
                         sem_in, sem_out, _T2, sid)

    plsc.subcore_barrier()

    @pl.when(cid == 0)
    def _():
        _gather_phase(mt_hbm, log_hbm, t1_hbm, out_hbm, (ids0, ids1),
                      (sel0, sel1), (idx0, idx1), (rows0, rows1),
                      sem, sem_in, sem_out, _T1, 0, sid)

    @pl.when(cid == 1)
    def _():
        _gather_phase(wt_hbm, log_hbm, t2_hbm, out_hbm, (ids0, ids1),
                      (sel0, sel1), (idx0, idx1), (rows0, rows1),
                      sem, sem_in, sem_out, _T2, _D, sid)


def kernel(log_seqs, time1_seqs, time2_seqs, month_pop_table, week_pop_table):
    log = log_seqs.reshape(_N).astype(jnp.int32)
    t1 = time1_seqs.reshape(_N).astype(jnp.int32)
    t2 = time2_seqs.reshape(_N).astype(jnp.int32)
    mtab = jnp.pad(month_pop_table, ((0, 0), (0, _VP - _V)))
    wtab = jnp.pad(week_pop_table, ((0, 0), (0, _VP - _V)))
    out = _popularity_gather(log, t1, t2, mtab, wtab)
    return out.reshape(_B, _L, 2 * _D)


# speedup vs baseline: 1.6556x; 1.0587x over previous
import functools

import jax
import jax.numpy as jnp
from jax import lax
from jax.experimental import pallas as pl
from jax.experimental.pallas import tpu as pltpu
from jax.experimental.pallas import tpu_sc as plsc

_B = 4096
_L = 200
_N = _B * _L
_V = 100001
_T1 = 12
_T2 = 5
_D = 16
_LANES = 16

_NC = 2
_NS = 16

_W = 128
_VP = 100096
_NCHUNK = _VP // _W
_NSLOT = 2 * ((_NCHUNK + 2 * _NS - 1) // (2 * _NS))

_TPT = _N // _NS
_M = 640
_GSTEPS = _TPT // _M


def _transpose_phase(tab_hbm, dst_hbm, in_bufs, out_bufs, sem_in, sem_out,
                     t_cnt, sid):
    nrows = t_cnt * _LANES
    iota = lax.broadcasted_iota(jnp.int32, (_LANES,), 0)

    nq = nrows // 4

    def start_in(k, b):
        for q in range(4):
            pltpu.async_copy(
                tab_hbm.at[pl.ds(q * nq, nq), pl.ds(k * _W, _W)],
                in_bufs[b].at[pl.ds(q * nq, nq), pl.ds(0, _W)], sem_in)

    def wait_in(k, b):
        for q in range(4):
            pltpu.make_async_copy(
                tab_hbm.at[pl.ds(q * nq, nq), pl.ds(k * _W, _W)],
                in_bufs[b].at[pl.ds(q * nq, nq), pl.ds(0, _W)], sem_in).wait()

    def out_slice(k):
        return dst_hbm.at[pl.ds(k * _W * t_cnt, _W * t_cnt)]

    def out_buf(b):
        return out_bufs[b].at[pl.ds(0, _W * t_cnt)]

    def transpose_chunk(b):
        in_stage, out_stage = in_bufs[b], out_bufs[b]

        def per_t(t, c1):
            rowidx = t * _LANES + iota
            for c0 in range(0, _W, 16):
                vs = [plsc.load_gather(
                    in_stage,
                    [rowidx, jnp.full((_LANES,), c0 + j, jnp.int32)])
                    for j in range(16)]
                for j in range(16):
                    out_stage[(c0 + j) * t_cnt + t, :] = vs[j]
            return c1

        lax.fori_loop(0, t_cnt, per_t, 0)

    start_in(sid, 0)

    def ring(kk2, c):
        for b in (0, 1):
            kk = kk2 * 2 + b
            k = sid + kk * _NS

            @pl.when(k < _NCHUNK)
            def _():
                wait_in(k, b)

                @pl.when(sid + (kk + 1) * _NS < _NCHUNK)
                def _():
                    start_in(sid + (kk + 1) * _NS, 1 - b)

                @pl.when(kk >= 2)
                def _():
                    pltpu.make_async_copy(
                        out_buf(b), out_slice(k), sem_out).wait()

                transpose_chunk(b)
                pltpu.async_copy(out_buf(b), out_slice(k), sem_out)

        return c

    lax.fori_loop(0, _NSLOT // 2, ring, 0)
    pltpu.make_async_copy(out_buf(0), out_slice(0), sem_out).wait()
    pltpu.make_async_copy(out_buf(1), out_slice(0), sem_out).wait()


def _gather_phase(src_hbm, ids_hbm, sel_hbm, out_hbm, ids_bufs, sel_bufs,
                  idx_bufs, row_bufs, sem_gs, sem_in, sem_out, t_cnt, col0,
                  sid):
    def start_in(m, b):
        base = sid * _TPT + m * _M
        pltpu.async_copy(ids_hbm.at[pl.ds(base, _M)], ids_bufs[b], sem_in)
        pltpu.async_copy(sel_hbm.at[pl.ds(base, _M)], sel_bufs[b], sem_in)

    def wait_in(m, b):
        base = sid * _TPT + m * _M
        pltpu.make_async_copy(
            ids_hbm.at[pl.ds(base, _M)], ids_bufs[b], sem_in).wait()
        pltpu.make_async_copy(
            sel_hbm.at[pl.ds(base, _M)], sel_bufs[b], sem_in).wait()

    def out_slice(m):
        base = sid * _TPT + m * _M
        return out_hbm.at[pl.ds(base, _M), pl.ds(col0, _D)]

    start_in(0, 0)

    def ring(m2, c):
        for b in (0, 1):
            m = m2 * 2 + b
            wait_in(m, b)

            @pl.when(m + 1 < _GSTEPS)
            def _():
                start_in(m + 1, 1 - b)

            def compute(i, c2):
                s = pl.ds(i * _LANES, _LANES)
                idx_bufs[b][s] = ids_bufs[b][s] * t_cnt + sel_bufs[b][s]
                return c2

            lax.fori_loop(0, _M // _LANES, compute, 0)

            @pl.when(m >= 2)
            def _():
                pltpu.make_async_copy(
                    row_bufs[b], out_slice(m), sem_out).wait()

            pltpu.async_copy(src_hbm.at[idx_bufs[b]], row_bufs[b], sem_gs[b])

            @pl.when(m >= 1)
            def _():
                pltpu.make_async_copy(
                    src_hbm.at[idx_bufs[1 - b]], row_bufs[1 - b],
                    sem_gs[1 - b]).wait()
                pltpu.async_copy(row_bufs[1 - b], out_slice(m - 1), sem_out)
        return c

    lax.fori_loop(0, _GSTEPS // 2, ring, 0)
    pltpu.make_async_copy(
        src_hbm.at[idx_bufs[1]], row_bufs[1], sem_gs[1]).wait()
    pltpu.make_async_copy(row_bufs[0], out_slice(0), sem_out).wait()
    pltpu.async_copy(row_bufs[1], out_slice(_GSTEPS - 1), sem_out)
    pltpu.make_async_copy(row_bufs[1], out_slice(0), sem_out).wait()


@functools.partial(
    pl.kernel,
    out_type=jax.ShapeDtypeStruct((_N, 2 * _D), jnp.float32),
    mesh=plsc.VectorSubcoreMesh(
        core_axis_name="c", subcore_axis_name="s",
        num_cores=_NC, num_subcores=_NS),
    compiler_params=pltpu.CompilerParams(
        use_tc_tiling_on_sc=False, needs_layout_passes=False),
    scratch_types=[
        pltpu.HBM((_VP * _T1, _D), jnp.float32),
        pltpu.HBM((_VP * _T2, _D), jnp.float32),
        pltpu.VMEM((_T1 * _LANES, _W + 1), jnp.float32),
        pltpu.VMEM((_T1 * _LANES, _W + 1), jnp.float32),
        pltpu.VMEM((_W * _T1, _D), jnp.float32),
        pltpu.VMEM((_W * _T1, _D), jnp.float32),
        pltpu.VMEM((_M,), jnp.int32),
        pltpu.VMEM((_M,), jnp.int32),
        pltpu.VMEM((_M,), jnp.int32),
        pltpu.VMEM((_M,), jnp.int32),
        pltpu.VMEM((_M,), jnp.int32),
        pltpu.VMEM((_M,), jnp.int32),
        pltpu.VMEM((_M, _D), jnp.float32),
        pltpu.VMEM((_M, _D), jnp.float32),
        pltpu.SemaphoreType.DMA,
        pltpu.SemaphoreType.DMA,
        pltpu.SemaphoreType.DMA,
        pltpu.SemaphoreType.DMA,
    ],
)
def _popularity_gather(log_hbm, t1_hbm, t2_hbm, mtab_hbm, wtab_hbm, out_hbm,
                       mt_hbm, wt_hbm, in0, in1, ost0, ost1,
                       ids0, ids1, sel0, sel1, idx0, idx1, rows0, rows1,
                       sem, sem_in, sem_out, sem_g1):
    cid = lax.axis_index("c")
    sid = lax.axis_index("s")

    @pl.when(cid == 0)
    def _():
        _transpose_phase(mtab_hbm, mt_hbm, (in0, in1), (ost0, ost1),
                         sem_in, sem_out, _T1, sid)

    @pl.when(cid == 1)
    def _():
        _transpose_phase(wtab_hbm, wt_hbm, (in0, in1), (ost0, ost1),
                         sem_in, sem_out, _T2, sid)

    plsc.subcore_barrier()

    @pl.when(cid == 0)
    def _():
        _gather_phase(mt_hbm, log_hbm, t1_hbm, out_hbm, (ids0, ids1),
                      (sel0, sel1), (idx0, idx1), (rows0, rows1),
                      (sem, sem_g1), sem_in, sem_out, _T1, 0, sid)

    @pl.when(cid == 1)
    def _():
        _gather_phase(wt_hbm, log_hbm, t2_hbm, out_hbm, (ids0, ids1),
                      (sel0, sel1), (idx0, idx1), (rows0, rows1),
                      (sem, sem_g1), sem_in, sem_out, _T2, _D, sid)


def kernel(log_seqs, time1_seqs, time2_seqs, month_pop_table, week_pop_table):
    log = log_seqs.reshape(_N).astype(jnp.int32)
    t1 = time1_seqs.reshape(_N).astype(jnp.int32)
    t2 = time2_seqs.reshape(_N).astype(jnp.int32)
    mtab = jnp.pad(month_pop_table, ((0, 0), (0, _VP - _V)))
    wtab = jnp.pad(week_pop_table, ((0, 0), (0, _VP - _V)))
    out = _popularity_gather(log, t1, t2, mtab, wtab)
    return out.reshape(_B, _L, 2 * _D)
